# baseline (speedup 1.0000x reference)
import functools

import jax
import jax.numpy as jnp
from jax import lax
from jax.experimental import pallas as pl
from jax.experimental.pallas import tpu as pltpu
from jax.experimental.pallas import tpu_sc as plsc

N = 100000
XS = 128
HS = 128

NC = 2
NS = 16
NW = NC * NS
CHUNK = 64
GRP = 5
NPAD = 102400
TOTCH = NPAD // CHUNK
ACH = 100
BCH = 0
MAXCH = max(ACH, BCH)
GROW = GRP * CHUNK

BLK = 800


def _gather_body(hc_hbm, idx0_hbm, idx1_hbm, hc0_hbm, hc1_hbm,
                 idx0_v, idx1_v, ring, sem_a, sem_b):
    cid = lax.axis_index("c")
    sid = lax.axis_index("s")
    wid = cid * NS + sid
    on_a = cid == 0
    chunkbase = jnp.where(on_a, sid * ACH, 16 * ACH + sid * BCH)
    ngrp2 = jnp.where(on_a, ACH // (2 * GRP), BCH // (2 * GRP))
    pltpu.sync_copy(idx0_hbm.at[wid], idx0_v)
    pltpu.sync_copy(idx1_hbm.at[wid], idx1_v)
    outbase = chunkbase * CHUNK

    def fire(idxv, g, half, hsem):
        return [pltpu.async_copy(hc_hbm.at[idxv.at[g * GRP + k]],
                                 ring.at[half].at[pl.ds(k * CHUNK, CHUNK)],
                                 hsem)
                for k in range(GRP)]

    for idxv, out in ((idx0_v, hc0_hbm), (idx1_v, hc1_hbm)):
        def grp_body(g2, carry, idxv=idxv, out=out):
            geven = 2 * g2
            cps_a = fire(idxv, geven, 0, sem_a)
            cps_b = fire(idxv, geven + 1, 1, sem_b)
            for cp in cps_a:
                cp.wait()
            pltpu.sync_copy(ring.at[0],
                            out.at[pl.ds(outbase + geven * GROW, GROW)])
            for cp in cps_b:
                cp.wait()
            pltpu.sync_copy(ring.at[1],
                            out.at[pl.ds(outbase + (geven + 1) * GROW, GROW)])
            return carry
        lax.fori_loop(0, ngrp2, grp_body, 0)


_gather_call = functools.partial(
    pl.kernel,
    mesh=plsc.VectorSubcoreMesh(core_axis_name="c", subcore_axis_name="s"),
    out_type=[jax.ShapeDtypeStruct((NPAD, HS), jnp.float32)] * 2,
    scratch_types=[
        pltpu.VMEM((MAXCH, CHUNK), jnp.int32),
        pltpu.VMEM((MAXCH, CHUNK), jnp.int32),
        pltpu.VMEM((2, GROW, HS), jnp.float32),
        pltpu.SemaphoreType.DMA,
        pltpu.SemaphoreType.DMA,
    ],
)(_gather_body)


PBLK = 2000


def _pack_body(h_ref, c_ref, out_ref):
    hb = h_ref[...].astype(jnp.bfloat16)
    cb = c_ref[...].astype(jnp.bfloat16)
    uh = lax.convert_element_type(
        lax.bitcast_convert_type(hb, jnp.uint16), jnp.uint32)
    uc = lax.convert_element_type(
        lax.bitcast_convert_type(cb, jnp.uint16), jnp.uint32)
    out_ref[...] = lax.bitcast_convert_type(uh | (uc << 16), jnp.float32)


def _pack_call(h, c):
    spec = pl.BlockSpec((PBLK, HS), lambda i: (i, 0))
    return pl.pallas_call(
        _pack_body,
        grid=(N // PBLK,),
        in_specs=[spec, spec],
        out_specs=spec,
        out_shape=jax.ShapeDtypeStruct((N, HS), jnp.float32),
        compiler_params=pltpu.CompilerParams(
            dimension_semantics=("arbitrary",)),
    )(h, c)


def _tc_body(x_ref, hc0_ref, hc1_ref,
             w_ref, u0_ref, u1_ref, uf0_ref, uf1_ref, biou_ref, bf_ref,
             hout_ref, cout_ref):
    dot = lambda a, b: lax.dot_general(a, b, (((1,), (0,)), ((), ())),
                                       preferred_element_type=jnp.float32)
    bft = jnp.bfloat16
    u0 = lax.bitcast_convert_type(hc0_ref[...], jnp.uint32)
    u1 = lax.bitcast_convert_type(hc1_ref[...], jnp.uint32)
    mask = jnp.uint32(0xFFFF0000)
    h0 = lax.bitcast_convert_type(u0 << 16, jnp.float32).astype(bft)
    c0 = lax.bitcast_convert_type(u0 & mask, jnp.float32)
    h1 = lax.bitcast_convert_type(u1 << 16, jnp.float32).astype(bft)
    c1 = lax.bitcast_convert_type(u1 & mask, jnp.float32)
    sg = lambda z: 0.5 + 0.5 * jnp.tanh(0.5 * z)
    iou = (dot(x_ref[...], w_ref[...]) + dot(h0, u0_ref[...])
           + dot(h1, u1_ref[...]) + biou_ref[...])
    f = sg(dot(h0, uf0_ref[...]) + dot(h1, uf1_ref[...]) + bf_ref[...])
    c_red = f[:, :HS] * c0 + f[:, HS:] * c1
    i = sg(iou[:, :HS])
    o = sg(iou[:, HS:2 * HS])
    u = jnp.tanh(iou[:, 2 * HS:])
    c_new = i * u + c_red
    cout_ref[...] = c_new
    hout_ref[...] = o * jnp.tanh(c_new)


def _tc_call(x, hc0, hc1, w, u0, u1, uf0, uf1, biou, bf):
    grid = (N // BLK,)
    row_spec = pl.BlockSpec((BLK, HS), lambda i: (i, 0))
    full = lambda a: pl.BlockSpec(a.shape, lambda i: (0,) * a.ndim)
    return pl.pallas_call(
        _tc_body,
        grid=grid,
        in_specs=[row_spec, row_spec, row_spec,
                  full(w), full(u0), full(u1), full(uf0), full(uf1),
                  full(biou), full(bf)],
        out_specs=[row_spec, row_spec],
        out_shape=[jax.ShapeDtypeStruct((N, HS), jnp.float32)] * 2,
        compiler_params=pltpu.CompilerParams(
            dimension_semantics=("arbitrary",)),
    )(x, hc0, hc1, w, u0, u1, uf0, uf1, biou, bf)


def _split_idx(col):
    flat = jnp.pad(col, (0, NPAD - N)).reshape(TOTCH, CHUNK)
    pa = jnp.pad(flat[:16 * ACH].reshape(16, ACH, CHUNK),
                 ((0, 0), (0, MAXCH - ACH), (0, 0)))
    if BCH:
        pb = jnp.pad(flat[16 * ACH:].reshape(16, BCH, CHUNK),
                     ((0, 0), (0, MAXCH - BCH), (0, 0)))
    else:
        pb = jnp.zeros((16, MAXCH, CHUNK), jnp.int32)
    return jnp.concatenate([pa, pb], axis=0)


def kernel(x, h, c, child_idx, W_iou, U_iou, b_iou, Uf_w, Uf_b):
    idx0 = _split_idx(child_idx[:, 0])
    idx1 = _split_idx(child_idx[:, 1])
    bft = jnp.bfloat16
    hc = _pack_call(h, c)
    hc0, hc1 = _gather_call(hc, idx0, idx1)
    h_new, c_new = _tc_call(
        x.astype(bft), hc0, hc1,
        W_iou.astype(bft), U_iou[:HS].astype(bft), U_iou[HS:].astype(bft),
        Uf_w[:HS].astype(bft), Uf_w[HS:].astype(bft),
        b_iou, Uf_b.reshape(1, 2 * HS))
    return (h_new, c_new)


# speedup vs baseline: 1.1585x; 1.1585x over previous
import functools

import jax
import jax.numpy as jnp
from jax import lax
from jax.experimental import pallas as pl
from jax.experimental.pallas import tpu as pltpu
from jax.experimental.pallas import tpu_sc as plsc

N = 100000
XS = 128
HS = 128

NC = 2
NS = 16
NW = NC * NS
CHUNK = 64
GRP = 5
NPAD = 102400
TOTCH = NPAD // CHUNK
ACH = 80
BCH = 20
MAXCH = max(ACH, BCH)
GROW = GRP * CHUNK

BLK = 1000


def _gather_body(hc_hbm, idx0_hbm, idx1_hbm, hc0_hbm, hc1_hbm,
                 idx0_v, idx1_v, ring, sem_a, sem_b):
    cid = lax.axis_index("c")
    sid = lax.axis_index("s")
    wid = cid * NS + sid
    on_a = cid == 0
    chunkbase = jnp.where(on_a, sid * ACH, 16 * ACH + sid * BCH)
    ngrp2 = jnp.where(on_a, ACH // (2 * GRP), BCH // (2 * GRP))
    pltpu.sync_copy(idx0_hbm.at[wid], idx0_v)
    pltpu.sync_copy(idx1_hbm.at[wid], idx1_v)
    outbase = chunkbase * CHUNK

    def fire(idxv, g, half, hsem):
        return [pltpu.async_copy(hc_hbm.at[idxv.at[g * GRP + k]],
                                 ring.at[half].at[pl.ds(k * CHUNK, CHUNK)],
                                 hsem)
                for k in range(GRP)]

    for idxv, out in ((idx0_v, hc0_hbm), (idx1_v, hc1_hbm)):
        def grp_body(g2, carry, idxv=idxv, out=out):
            geven = 2 * g2
            cps_a = fire(idxv, geven, 0, sem_a)
            cps_b = fire(idxv, geven + 1, 1, sem_b)
            for cp in cps_a:
                cp.wait()
            pltpu.sync_copy(ring.at[0],
                            out.at[pl.ds(outbase + geven * GROW, GROW)])
            for cp in cps_b:
                cp.wait()
            pltpu.sync_copy(ring.at[1],
                            out.at[pl.ds(outbase + (geven + 1) * GROW, GROW)])
            return carry
        lax.fori_loop(0, ngrp2, grp_body, 0)


_gather_call = functools.partial(
    pl.kernel,
    mesh=plsc.VectorSubcoreMesh(core_axis_name="c", subcore_axis_name="s"),
    out_type=[jax.ShapeDtypeStruct((NPAD, HS), jnp.float32)] * 2,
    scratch_types=[
        pltpu.VMEM((MAXCH, CHUNK), jnp.int32),
        pltpu.VMEM((MAXCH, CHUNK), jnp.int32),
        pltpu.VMEM((2, GROW, HS), jnp.float32),
        pltpu.SemaphoreType.DMA,
        pltpu.SemaphoreType.DMA,
    ],
)(_gather_body)


PBLK = 2000


def _pack_body(h_ref, c_ref, out_ref):
    hb = h_ref[...].astype(jnp.bfloat16)
    cb = c_ref[...].astype(jnp.bfloat16)
    uh = lax.convert_element_type(
        lax.bitcast_convert_type(hb, jnp.uint16), jnp.uint32)
    uc = lax.convert_element_type(
        lax.bitcast_convert_type(cb, jnp.uint16), jnp.uint32)
    out_ref[...] = lax.bitcast_convert_type(uh | (uc << 16), jnp.float32)


def _pack_call(h, c):
    spec = pl.BlockSpec((PBLK, HS), lambda i: (i, 0))
    return pl.pallas_call(
        _pack_body,
        grid=(N // PBLK,),
        in_specs=[spec, spec],
        out_specs=spec,
        out_shape=jax.ShapeDtypeStruct((N, HS), jnp.float32),
        compiler_params=pltpu.CompilerParams(
            dimension_semantics=("arbitrary",)),
    )(h, c)


def _tc_body(x_ref, hc0_ref, hc1_ref,
             w_ref, u0_ref, u1_ref, biou_ref, bf_ref,
             hout_ref, cout_ref):
    dot = lambda a, b: lax.dot_general(a, b, (((1,), (0,)), ((), ())),
                                       preferred_element_type=jnp.float32)
    bft = jnp.bfloat16
    u0 = lax.bitcast_convert_type(hc0_ref[...], jnp.uint32)
    u1 = lax.bitcast_convert_type(hc1_ref[...], jnp.uint32)
    mask = jnp.uint32(0xFFFF0000)
    h0 = lax.bitcast_convert_type(u0 << 16, jnp.float32).astype(bft)
    c0 = lax.bitcast_convert_type(u0 & mask, jnp.float32)
    h1 = lax.bitcast_convert_type(u1 << 16, jnp.float32).astype(bft)
    c1 = lax.bitcast_convert_type(u1 & mask, jnp.float32)
    sg = lambda z: 0.5 + 0.5 * jnp.tanh(0.5 * z)
    r = dot(h0, u0_ref[...]) + dot(h1, u1_ref[...])
    iou = dot(x_ref[...], w_ref[...]) + r[:, :3 * HS] + biou_ref[...]
    f = sg(r[:, 3 * HS:] + bf_ref[...])
    c_red = f[:, :HS] * c0 + f[:, HS:] * c1
    i = sg(iou[:, :HS])
    o = sg(iou[:, HS:2 * HS])
    u = jnp.tanh(iou[:, 2 * HS:])
    c_new = i * u + c_red
    cout_ref[...] = c_new
    hout_ref[...] = o * jnp.tanh(c_new)


def _tc_call(x, hc0, hc1, w, u0, u1, biou, bf):
    grid = (N // BLK,)
    row_spec = pl.BlockSpec((BLK, HS), lambda i: (i, 0))
    full = lambda a: pl.BlockSpec(a.shape, lambda i: (0,) * a.ndim)
    return pl.pallas_call(
        _tc_body,
        grid=grid,
        in_specs=[row_spec, row_spec, row_spec,
                  full(w), full(u0), full(u1), full(biou), full(bf)],
        out_specs=[row_spec, row_spec],
        out_shape=[jax.ShapeDtypeStruct((N, HS), jnp.float32)] * 2,
        compiler_params=pltpu.CompilerParams(
            dimension_semantics=("arbitrary",)),
    )(x, hc0, hc1, w, u0, u1, biou, bf)


def _split_idx(col):
    flat = jnp.pad(col, (0, NPAD - N)).reshape(TOTCH, CHUNK)
    pa = jnp.pad(flat[:16 * ACH].reshape(16, ACH, CHUNK),
                 ((0, 0), (0, MAXCH - ACH), (0, 0)))
    if BCH:
        pb = jnp.pad(flat[16 * ACH:].reshape(16, BCH, CHUNK),
                     ((0, 0), (0, MAXCH - BCH), (0, 0)))
    else:
        pb = jnp.zeros((16, MAXCH, CHUNK), jnp.int32)
    return jnp.concatenate([pa, pb], axis=0)


def kernel(x, h, c, child_idx, W_iou, U_iou, b_iou, Uf_w, Uf_b):
    idx0 = _split_idx(child_idx[:, 0])
    idx1 = _split_idx(child_idx[:, 1])
    bft = jnp.bfloat16
    hc = _pack_call(h, c)
    hc0, hc1 = _gather_call(hc, idx0, idx1)
    u0 = jnp.concatenate([U_iou[:HS], Uf_w[:HS]], axis=1).astype(bft)
    u1 = jnp.concatenate([U_iou[HS:], Uf_w[HS:]], axis=1).astype(bft)
    h_new, c_new = _tc_call(
        x.astype(bft), hc0, hc1,
        W_iou.astype(bft), u0, u1,
        b_iou, Uf_b.reshape(1, 2 * HS))
    return (h_new, c_new)


# speedup vs baseline: 1.2115x; 1.0458x over previous
import functools

import jax
import jax.numpy as jnp
from jax import lax
from jax.experimental import pallas as pl
from jax.experimental.pallas import tpu as pltpu
from jax.experimental.pallas import tpu_sc as plsc

N = 100000
XS = 128
HS = 128

NC = 2
NS = 16
NW = NC * NS
CHUNK = 64
GRP = 5
NPAD = 102400
TOTCH = NPAD // CHUNK
ACH = 90
BCH = 10
MAXCH = max(ACH, BCH)
GROW = GRP * CHUNK

BLK = 1000


def _gather_body(hc_hbm, idx0_hbm, idx1_hbm, hc0_hbm, hc1_hbm,
                 idx0_v, idx1_v, ring, sem_a, sem_b):
    cid = lax.axis_index("c")
    sid = lax.axis_index("s")
    wid = cid * NS + sid
    on_a = cid == 0
    chunkbase = jnp.where(on_a, sid * ACH, 16 * ACH + sid * BCH)
    ngrp2 = jnp.where(on_a, ACH // (2 * GRP), BCH // (2 * GRP))
    pltpu.sync_copy(idx0_hbm.at[wid], idx0_v)
    pltpu.sync_copy(idx1_hbm.at[wid], idx1_v)
    outbase = chunkbase * CHUNK

    def fire(idxv, g, half, hsem):
        return [pltpu.async_copy(hc_hbm.at[idxv.at[g * GRP + k]],
                                 ring.at[half].at[pl.ds(k * CHUNK, CHUNK)],
                                 hsem)
                for k in range(GRP)]

    for idxv, out in ((idx0_v, hc0_hbm), (idx1_v, hc1_hbm)):
        def grp_body(g2, carry, idxv=idxv, out=out):
            geven = 2 * g2
            cps_a = fire(idxv, geven, 0, sem_a)
            cps_b = fire(idxv, geven + 1, 1, sem_b)
            for cp in cps_a:
                cp.wait()
            pltpu.sync_copy(ring.at[0],
                            out.at[pl.ds(outbase + geven * GROW, GROW)])
            for cp in cps_b:
                cp.wait()
            pltpu.sync_copy(ring.at[1],
                            out.at[pl.ds(outbase + (geven + 1) * GROW, GROW)])
            return carry
        lax.fori_loop(0, ngrp2, grp_body, 0)


_gather_call = functools.partial(
    pl.kernel,
    mesh=plsc.VectorSubcoreMesh(core_axis_name="c", subcore_axis_name="s"),
    out_type=[jax.ShapeDtypeStruct((NPAD, HS), jnp.float32)] * 2,
    scratch_types=[
        pltpu.VMEM((MAXCH, CHUNK), jnp.int32),
        pltpu.VMEM((MAXCH, CHUNK), jnp.int32),
        pltpu.VMEM((2, GROW, HS), jnp.float32),
        pltpu.SemaphoreType.DMA,
        pltpu.SemaphoreType.DMA,
    ],
)(_gather_body)


PBLK = 2000


def _pack_body(h_ref, c_ref, out_ref):
    hb = h_ref[...].astype(jnp.bfloat16)
    cb = c_ref[...].astype(jnp.bfloat16)
    uh = lax.convert_element_type(
        lax.bitcast_convert_type(hb, jnp.uint16), jnp.uint32)
    uc = lax.convert_element_type(
        lax.bitcast_convert_type(cb, jnp.uint16), jnp.uint32)
    out_ref[...] = lax.bitcast_convert_type(uh | (uc << 16), jnp.float32)


def _pack_call(h, c):
    spec = pl.BlockSpec((PBLK, HS), lambda i: (i, 0))
    return pl.pallas_call(
        _pack_body,
        grid=(N // PBLK,),
        in_specs=[spec, spec],
        out_specs=spec,
        out_shape=jax.ShapeDtypeStruct((N, HS), jnp.float32),
        compiler_params=pltpu.CompilerParams(
            dimension_semantics=("arbitrary",)),
    )(h, c)


def _tc_body(x_ref, hc0_ref, hc1_ref,
             w_ref, u0_ref, u1_ref, biou_ref, bf_ref,
             hout_ref, cout_ref):
    dot = lambda a, b: lax.dot_general(a, b, (((1,), (0,)), ((), ())),
                                       preferred_element_type=jnp.float32)
    bft = jnp.bfloat16
    u0 = lax.bitcast_convert_type(hc0_ref[...], jnp.uint32)
    u1 = lax.bitcast_convert_type(hc1_ref[...], jnp.uint32)
    mask = jnp.uint32(0xFFFF0000)
    h0 = lax.bitcast_convert_type(u0 << 16, jnp.float32).astype(bft)
    c0 = lax.bitcast_convert_type(u0 & mask, jnp.float32)
    h1 = lax.bitcast_convert_type(u1 << 16, jnp.float32).astype(bft)
    c1 = lax.bitcast_convert_type(u1 & mask, jnp.float32)
    sg = lambda z: 0.5 + 0.5 * jnp.tanh(0.5 * z)
    r = dot(h0, u0_ref[...]) + dot(h1, u1_ref[...])
    iou = dot(x_ref[...], w_ref[...]) + r[:, :3 * HS] + biou_ref[...]
    f = sg(r[:, 3 * HS:] + bf_ref[...])
    c_red = f[:, :HS] * c0 + f[:, HS:] * c1
    i = sg(iou[:, :HS])
    o = sg(iou[:, HS:2 * HS])
    u = jnp.tanh(iou[:, 2 * HS:])
    c_new = i * u + c_red
    cout_ref[...] = c_new
    hout_ref[...] = o * jnp.tanh(c_new)


def _tc_call(x, hc0, hc1, w, u0, u1, biou, bf):
    grid = (N // BLK,)
    row_spec = pl.BlockSpec((BLK, HS), lambda i: (i, 0))
    full = lambda a: pl.BlockSpec(a.shape, lambda i: (0,) * a.ndim)
    return pl.pallas_call(
        _tc_body,
        grid=grid,
        in_specs=[row_spec, row_spec, row_spec,
                  full(w), full(u0), full(u1), full(biou), full(bf)],
        out_specs=[row_spec, row_spec],
        out_shape=[jax.ShapeDtypeStruct((N, HS), jnp.float32)] * 2,
        compiler_params=pltpu.CompilerParams(
            dimension_semantics=("arbitrary",)),
    )(x, hc0, hc1, w, u0, u1, biou, bf)


def _split_idx(col):
    flat = jnp.pad(col, (0, NPAD - N)).reshape(TOTCH, CHUNK)
    pa = jnp.pad(flat[:16 * ACH].reshape(16, ACH, CHUNK),
                 ((0, 0), (0, MAXCH - ACH), (0, 0)))
    if BCH:
        pb = jnp.pad(flat[16 * ACH:].reshape(16, BCH, CHUNK),
                     ((0, 0), (0, MAXCH - BCH), (0, 0)))
    else:
        pb = jnp.zeros((16, MAXCH, CHUNK), jnp.int32)
    return jnp.concatenate([pa, pb], axis=0)


def kernel(x, h, c, child_idx, W_iou, U_iou, b_iou, Uf_w, Uf_b):
    idx0 = _split_idx(child_idx[:, 0])
    idx1 = _split_idx(child_idx[:, 1])
    bft = jnp.bfloat16
    hc = _pack_call(h, c)
    hc0, hc1 = _gather_call(hc, idx0, idx1)
    u0 = jnp.concatenate([U_iou[:HS], Uf_w[:HS]], axis=1).astype(bft)
    u1 = jnp.concatenate([U_iou[HS:], Uf_w[HS:]], axis=1).astype(bft)
    h_new, c_new = _tc_call(
        x.astype(bft), hc0, hc1,
        W_iou.astype(bft), u0, u1,
        b_iou, Uf_b.reshape(1, 2 * HS))
    return (h_new, c_new)
